# Initial kernel scaffold; baseline (speedup 1.0000x reference)
import jax
import jax.numpy as jnp
from jax.experimental import pallas as pl


def kernel(x, edge_index, W_l, W_r):
    raise NotImplementedError("write your pallas kernel here")


# speedup vs baseline: 54.2945x; 54.2945x over previous
import functools

import jax
import jax.numpy as jnp
from jax import lax
from jax.experimental import pallas as pl
from jax.experimental.pallas import tpu as pltpu
from jax.experimental.pallas import tpu_sc as plsc

N_NODES = 99999
N_EDGES = N_NODES * 64
N_OUT = N_NODES // 3

NC, NS = 2, 16
NW = NC * NS

CHUNK = 8000
VECS = CHUNK // 16
CHUNKS_PER_TILE = 25
LAST_SLOT = NW * CHUNKS_PER_TILE - 1
TAIL_BASE = N_EDGES - CHUNK
TAIL_SKIP_VECS = (NW * CHUNKS_PER_TILE * CHUNK - N_EDGES) // 16

XPAD = N_NODES + 1
XWORDS = XPAD // 2
SLICE = 1056
OUTP = NW * SLICE

_mesh = plsc.VectorSubcoreMesh(
    core_axis_name="c", subcore_axis_name="s", num_cores=NC, num_subcores=NS
)
_params = pltpu.CompilerParams(needs_layout_passes=False)


@functools.partial(
    pl.kernel,
    out_type=jax.ShapeDtypeStruct((NW * OUTP,), jnp.float32),
    mesh=_mesh,
    compiler_params=_params,
    scratch_types=[
        pltpu.VMEM((XWORDS,), jnp.int32),
        pltpu.VMEM((OUTP,), jnp.float32),
        pltpu.VMEM((CHUNK,), jnp.int32),
        pltpu.VMEM((CHUNK,), jnp.int32),
    ],
)
def _edge_accumulate(xpk_hbm, ei_hbm, parts_hbm, xpk_v, acc_v, src_v, dst_v):
    wid = lax.axis_index("s") * NC + lax.axis_index("c")
    pltpu.sync_copy(xpk_hbm, xpk_v)

    def _zero(i, c):
        acc_v[pl.ds(i * 16, 16)] = jnp.zeros((16,), jnp.float32)
        return c

    lax.fori_loop(0, OUTP // 16, _zero, 0)

    third = jnp.float32(1.0 / 3.0)

    def _chunk(ci, c):
        slot = wid * CHUNKS_PER_TILE + ci
        is_tail = slot == LAST_SLOT
        base = jnp.where(is_tail, TAIL_BASE, slot * CHUNK)
        j0 = jnp.where(is_tail, TAIL_SKIP_VECS, 0)
        pltpu.sync_copy(ei_hbm.at[pl.ds(base, CHUNK)], src_v)
        pltpu.sync_copy(ei_hbm.at[pl.ds(N_EDGES + base, CHUNK)], dst_v)

        def _vec(j, cc):
            o = j * 16
            s = src_v[pl.ds(o, 16)]
            d = dst_v[pl.ds(o, 16)]
            w = plsc.load_gather(xpk_v, [jnp.right_shift(s, 1)])
            sh = jnp.left_shift(jnp.bitwise_and(s, 1), 4)
            bits = jnp.left_shift(lax.shift_right_logical(w, sh), 16)
            val = plsc.bitcast(bits, jnp.float32)
            q = (d.astype(jnp.float32) * third).astype(jnp.int32)
            keep = (q * 3) == d
            plsc.addupdate_scatter(acc_v, [q], val, mask=keep)
            return cc

        lax.fori_loop(j0, VECS, _vec, 0)
        return c

    lax.fori_loop(0, CHUNKS_PER_TILE, _chunk, 0)
    pltpu.sync_copy(acc_v, parts_hbm.at[pl.ds(wid * OUTP, OUTP)])


@functools.partial(
    pl.kernel,
    out_type=jax.ShapeDtypeStruct((OUTP,), jnp.float32),
    mesh=_mesh,
    compiler_params=_params,
    scratch_types=[
        pltpu.VMEM((NW * SLICE,), jnp.float32),
        pltpu.VMEM((SLICE,), jnp.float32),
        pltpu.VMEM((16,), jnp.float32),
        pltpu.VMEM((16,), jnp.float32),
        pltpu.VMEM((SLICE,), jnp.float32),
        pltpu.SemaphoreType.DMA,
    ],
)
def _combine(parts_hbm, x3_hbm, wl_hbm, wr_hbm, out_hbm,
             rows_v, x3_v, wl_v, wr_v, o_v, sem):
    wid = lax.axis_index("s") * NC + lax.axis_index("c")
    base = wid * SLICE
    cps = [
        pltpu.async_copy(
            parts_hbm.at[pl.ds(r * OUTP + base, SLICE)],
            rows_v.at[pl.ds(r * SLICE, SLICE)],
            sem,
        )
        for r in range(NW)
    ]
    pltpu.sync_copy(x3_hbm.at[pl.ds(base, SLICE)], x3_v)
    pltpu.sync_copy(wl_hbm, wl_v)
    pltpu.sync_copy(wr_hbm, wr_v)
    for cp in cps:
        cp.wait()
    wl = wl_v[...]
    wr = wr_v[...]

    def _red(j, c):
        o = j * 16
        tot = rows_v[pl.ds(o, 16)]
        for r in range(1, NW):
            tot = tot + rows_v[pl.ds(r * SLICE + o, 16)]
        o_v[pl.ds(o, 16)] = tot * wl + x3_v[pl.ds(o, 16)] * wr
        return c

    lax.fori_loop(0, SLICE // 16, _red, 0)
    pltpu.sync_copy(o_v, out_hbm.at[pl.ds(base, SLICE)])


def kernel(x, edge_index, W_l, W_r):
    xf = x.reshape(-1)
    xb = jnp.concatenate(
        [xf.astype(jnp.bfloat16), jnp.zeros((XPAD - N_NODES,), jnp.bfloat16)]
    )
    xpk = lax.bitcast_convert_type(xb.reshape(XWORDS, 2), jnp.int32)
    x3 = jnp.concatenate([xf[::3], jnp.zeros((OUTP - N_OUT,), jnp.float32)])
    wl = jnp.full((16,), W_l[0, 0], jnp.float32)
    wr = jnp.full((16,), W_r[0, 0], jnp.float32)
    ei = edge_index.astype(jnp.int32).reshape(-1)
    parts = _edge_accumulate(xpk, ei)
    outp = _combine(parts, x3, wl, wr)
    return outp[:N_OUT]


# speedup vs baseline: 230.2736x; 4.2412x over previous
import functools

import jax
import jax.numpy as jnp
from jax import lax
from jax.experimental import pallas as pl
from jax.experimental.pallas import tpu as pltpu
from jax.experimental.pallas import tpu_sc as plsc

N_NODES = 99999
N_EDGES = N_NODES * 64
N_OUT = N_NODES // 3

NC, NS = 2, 16
NW = NC * NS

CHUNK = 8192
VECS = CHUNK // 16
NSLOTS = N_EDGES // CHUNK
CHUNKS_PER_TILE = (NSLOTS + NW - 1) // NW
TAIL = N_EDGES - NSLOTS * CHUNK
TAIL_VECS = TAIL // 16

XPAD = N_NODES + 1
XWORDS = XPAD // 2
SLICE = 1056
OUTP = NW * SLICE

_mesh = plsc.VectorSubcoreMesh(
    core_axis_name="c", subcore_axis_name="s", num_cores=NC, num_subcores=NS
)
_params = pltpu.CompilerParams(needs_layout_passes=False)


@functools.partial(
    pl.kernel,
    out_type=jax.ShapeDtypeStruct((NW * OUTP,), jnp.float32),
    mesh=_mesh,
    compiler_params=_params,
    scratch_types=[
        pltpu.VMEM((XWORDS,), jnp.int32),
        pltpu.VMEM((OUTP,), jnp.float32),
        pltpu.VMEM((2, CHUNK), jnp.int32),
        pltpu.VMEM((TAIL,), jnp.int32),
        pltpu.VMEM((TAIL,), jnp.int32),
    ],
)
def _edge_accumulate(xpk_hbm, ei_hbm, tsrc_hbm, tdst_hbm, parts_hbm,
                     xpk_v, acc_v, edge_v, tsrc_v, tdst_v):
    wid = lax.axis_index("s") * NC + lax.axis_index("c")
    pltpu.sync_copy(xpk_hbm, xpk_v)

    def _zero(i, c):
        acc_v[pl.ds(i * 16, 16)] = jnp.zeros((16,), jnp.float32)
        return c

    lax.fori_loop(0, OUTP // 16, _zero, 0)

    third = jnp.float32(1.0 / 3.0)

    def _body(s, d):
        w = plsc.load_gather(xpk_v, [jnp.right_shift(s, 1)])
        sh = jnp.left_shift(jnp.bitwise_and(s, 1), 4)
        bits = jnp.left_shift(lax.shift_right_logical(w, sh), 16)
        val = plsc.bitcast(bits, jnp.float32)
        q = (d.astype(jnp.float32) * third).astype(jnp.int32)
        keep = (q * 3) == d
        plsc.addupdate_scatter(acc_v, [q], val, mask=keep)

    def _chunk(ci, c):
        slot = ci * NW + wid

        @pl.when(slot < NSLOTS)
        def _():
            base = pl.multiple_of(slot * CHUNK, CHUNK)
            pltpu.sync_copy(ei_hbm.at[:, pl.ds(base, CHUNK)], edge_v)

            def _vec(j, cc):
                o = j * 16
                _body(edge_v[0, pl.ds(o, 16)], edge_v[1, pl.ds(o, 16)])
                return cc

            lax.fori_loop(0, VECS, _vec, 0)

        return c

    lax.fori_loop(0, CHUNKS_PER_TILE, _chunk, 0)

    @pl.when(wid == NW - 1)
    def _():
        pltpu.sync_copy(tsrc_hbm, tsrc_v)
        pltpu.sync_copy(tdst_hbm, tdst_v)

        def _tvec(j, cc):
            o = j * 16
            _body(tsrc_v[pl.ds(o, 16)], tdst_v[pl.ds(o, 16)])
            return cc

        lax.fori_loop(0, TAIL_VECS, _tvec, 0)

    pltpu.sync_copy(acc_v, parts_hbm.at[pl.ds(wid * OUTP, OUTP)])


@functools.partial(
    pl.kernel,
    out_type=jax.ShapeDtypeStruct((OUTP,), jnp.float32),
    mesh=_mesh,
    compiler_params=_params,
    scratch_types=[
        pltpu.VMEM((NW * SLICE,), jnp.float32),
        pltpu.VMEM((SLICE,), jnp.float32),
        pltpu.VMEM((16,), jnp.float32),
        pltpu.VMEM((16,), jnp.float32),
        pltpu.VMEM((SLICE,), jnp.float32),
        pltpu.SemaphoreType.DMA,
    ],
)
def _combine(parts_hbm, x3_hbm, wl_hbm, wr_hbm, out_hbm,
             rows_v, x3_v, wl_v, wr_v, o_v, sem):
    wid = lax.axis_index("s") * NC + lax.axis_index("c")
    base = wid * SLICE
    cps = [
        pltpu.async_copy(
            parts_hbm.at[pl.ds(r * OUTP + base, SLICE)],
            rows_v.at[pl.ds(r * SLICE, SLICE)],
            sem,
        )
        for r in range(NW)
    ]
    pltpu.sync_copy(x3_hbm.at[pl.ds(base, SLICE)], x3_v)
    pltpu.sync_copy(wl_hbm, wl_v)
    pltpu.sync_copy(wr_hbm, wr_v)
    for cp in cps:
        cp.wait()
    wl = wl_v[...]
    wr = wr_v[...]

    def _red(j, c):
        o = j * 16
        tot = rows_v[pl.ds(o, 16)]
        for r in range(1, NW):
            tot = tot + rows_v[pl.ds(r * SLICE + o, 16)]
        o_v[pl.ds(o, 16)] = tot * wl + x3_v[pl.ds(o, 16)] * wr
        return c

    lax.fori_loop(0, SLICE // 16, _red, 0)
    pltpu.sync_copy(o_v, out_hbm.at[pl.ds(base, SLICE)])


def kernel(x, edge_index, W_l, W_r):
    xf = x.reshape(-1)
    xb = jnp.concatenate(
        [xf.astype(jnp.bfloat16), jnp.zeros((XPAD - N_NODES,), jnp.bfloat16)]
    )
    xpk = lax.bitcast_convert_type(xb.reshape(XWORDS, 2), jnp.int32)
    x3 = jnp.concatenate([xf[::3], jnp.zeros((OUTP - N_OUT,), jnp.float32)])
    wl = jnp.full((16,), W_l[0, 0], jnp.float32)
    wr = jnp.full((16,), W_r[0, 0], jnp.float32)
    ei = edge_index.astype(jnp.int32)
    tsrc = ei[0, N_EDGES - TAIL:]
    tdst = ei[1, N_EDGES - TAIL:]
    parts = _edge_accumulate(xpk, ei, tsrc, tdst)
    outp = _combine(parts, x3, wl, wr)
    return outp[:N_OUT]


# speedup vs baseline: 257.8680x; 1.1198x over previous
import functools

import jax
import jax.numpy as jnp
from jax import lax
from jax.experimental import pallas as pl
from jax.experimental.pallas import tpu as pltpu
from jax.experimental.pallas import tpu_sc as plsc

N_NODES = 99999
N_EDGES = N_NODES * 64
N_OUT = N_NODES // 3

NC, NS = 2, 16
NW = NC * NS

CHUNK = 8192
VECS = CHUNK // 16
NSLOTS = N_EDGES // CHUNK
CHUNKS_PER_TILE = (NSLOTS + NW - 1) // NW
TAIL = N_EDGES - NSLOTS * CHUNK
TAIL_VECS = TAIL // 16

XPAD = N_NODES + 1
XWORDS = XPAD // 2
SLICE = 1056
OUTP = NW * SLICE

_mesh = plsc.VectorSubcoreMesh(
    core_axis_name="c", subcore_axis_name="s", num_cores=NC, num_subcores=NS
)
_params = pltpu.CompilerParams(needs_layout_passes=False)


@functools.partial(
    pl.kernel,
    out_type=jax.ShapeDtypeStruct((NW * OUTP,), jnp.float32),
    mesh=_mesh,
    compiler_params=_params,
    scratch_types=[
        pltpu.VMEM((XWORDS,), jnp.int32),
        pltpu.VMEM((OUTP,), jnp.float32),
        pltpu.VMEM((2, CHUNK), jnp.int32),
        pltpu.VMEM((2, CHUNK), jnp.int32),
        pltpu.VMEM((TAIL,), jnp.int32),
        pltpu.VMEM((TAIL,), jnp.int32),
        pltpu.SemaphoreType.DMA,
        pltpu.SemaphoreType.DMA,
    ],
)
def _edge_accumulate(xpk_hbm, ei_hbm, tsrc_hbm, tdst_hbm, parts_hbm,
                     xpk_v, acc_v, eb0, eb1, tsrc_v, tdst_v, sem0, sem1):
    wid = lax.axis_index("s") * NC + lax.axis_index("c")
    pltpu.sync_copy(xpk_hbm, xpk_v)

    def _zero(i, c):
        acc_v[pl.ds(i * 16, 16)] = jnp.zeros((16,), jnp.float32)
        return c

    lax.fori_loop(0, OUTP // 16, _zero, 0, unroll=8)

    third = jnp.float32(1.0 / 3.0)

    def _body(s, d):
        w = plsc.load_gather(xpk_v, [jnp.right_shift(s, 1)])
        sh = jnp.left_shift(jnp.bitwise_and(s, 1), 4)
        bits = jnp.left_shift(lax.shift_right_logical(w, sh), 16)
        val = plsc.bitcast(bits, jnp.float32)
        q = (d.astype(jnp.float32) * third).astype(jnp.int32)
        keep = (q * 3) == d
        plsc.addupdate_scatter(acc_v, [q], val, mask=keep)

    def _start(slot, buf, sem):
        base = pl.multiple_of(slot * CHUNK, CHUNK)
        pltpu.async_copy(ei_hbm.at[:, pl.ds(base, CHUNK)], buf, sem)

    bufs = (eb0, eb1)
    sems = (sem0, sem1)
    _start(wid, eb0, sem0)
    for ci in range(CHUNKS_PER_TILE):
        cur, csem = bufs[ci % 2], sems[ci % 2]
        if ci + 1 < CHUNKS_PER_TILE:
            nslot = (ci + 1) * NW + wid

            @pl.when(nslot < NSLOTS)
            def _(nslot=nslot, ci=ci):
                _start(nslot, bufs[(ci + 1) % 2], sems[(ci + 1) % 2])

        @pl.when(ci * NW + wid < NSLOTS)
        def _(cur=cur, csem=csem):
            pltpu.make_async_copy(
                ei_hbm.at[:, pl.ds(0, CHUNK)], cur, csem
            ).wait()

            def _vec(j, cc):
                o = j * 16
                _body(cur[0, pl.ds(o, 16)], cur[1, pl.ds(o, 16)])
                return cc

            lax.fori_loop(0, VECS, _vec, 0, unroll=4)

    @pl.when(wid == NW - 1)
    def _():
        pltpu.sync_copy(tsrc_hbm, tsrc_v)
        pltpu.sync_copy(tdst_hbm, tdst_v)

        def _tvec(j, cc):
            o = j * 16
            _body(tsrc_v[pl.ds(o, 16)], tdst_v[pl.ds(o, 16)])
            return cc

        lax.fori_loop(0, TAIL_VECS, _tvec, 0, unroll=4)

    pltpu.sync_copy(acc_v, parts_hbm.at[pl.ds(wid * OUTP, OUTP)])


@functools.partial(
    pl.kernel,
    out_type=jax.ShapeDtypeStruct((OUTP,), jnp.float32),
    mesh=_mesh,
    compiler_params=_params,
    scratch_types=[
        pltpu.VMEM((NW * SLICE,), jnp.float32),
        pltpu.VMEM((SLICE,), jnp.float32),
        pltpu.VMEM((16,), jnp.float32),
        pltpu.VMEM((16,), jnp.float32),
        pltpu.VMEM((SLICE,), jnp.float32),
        pltpu.SemaphoreType.DMA,
    ],
)
def _combine(parts_hbm, x3_hbm, wl_hbm, wr_hbm, out_hbm,
             rows_v, x3_v, wl_v, wr_v, o_v, sem):
    wid = lax.axis_index("s") * NC + lax.axis_index("c")
    base = wid * SLICE
    cps = [
        pltpu.async_copy(
            parts_hbm.at[pl.ds(r * OUTP + base, SLICE)],
            rows_v.at[pl.ds(r * SLICE, SLICE)],
            sem,
        )
        for r in range(NW)
    ]
    pltpu.sync_copy(x3_hbm.at[pl.ds(base, SLICE)], x3_v)
    pltpu.sync_copy(wl_hbm, wl_v)
    pltpu.sync_copy(wr_hbm, wr_v)
    for cp in cps:
        cp.wait()
    wl = wl_v[...]
    wr = wr_v[...]

    def _red(j, c):
        o = j * 16
        tot = rows_v[pl.ds(o, 16)]
        for r in range(1, NW):
            tot = tot + rows_v[pl.ds(r * SLICE + o, 16)]
        o_v[pl.ds(o, 16)] = tot * wl + x3_v[pl.ds(o, 16)] * wr
        return c

    lax.fori_loop(0, SLICE // 16, _red, 0)
    pltpu.sync_copy(o_v, out_hbm.at[pl.ds(base, SLICE)])


def kernel(x, edge_index, W_l, W_r):
    xf = x.reshape(-1)
    xb = jnp.concatenate(
        [xf.astype(jnp.bfloat16), jnp.zeros((XPAD - N_NODES,), jnp.bfloat16)]
    )
    xpk = lax.bitcast_convert_type(xb.reshape(XWORDS, 2), jnp.int32)
    x3 = jnp.concatenate([xf[::3], jnp.zeros((OUTP - N_OUT,), jnp.float32)])
    wl = jnp.full((16,), W_l[0, 0], jnp.float32)
    wr = jnp.full((16,), W_r[0, 0], jnp.float32)
    ei = edge_index.astype(jnp.int32)
    tsrc = ei[0, N_EDGES - TAIL:]
    tdst = ei[1, N_EDGES - TAIL:]
    parts = _edge_accumulate(xpk, ei, tsrc, tdst)
    outp = _combine(parts, x3, wl, wr)
    return outp[:N_OUT]


# speedup vs baseline: 553.7823x; 2.1475x over previous
import functools

import jax
import jax.numpy as jnp
from jax import lax
from jax.experimental import pallas as pl
from jax.experimental.pallas import tpu as pltpu
from jax.experimental.pallas import tpu_sc as plsc

N_NODES = 99999
N_EDGES = N_NODES * 64
N_OUT = N_NODES // 3

NC, NS = 2, 16
NW = NC * NS

CHUNK = 8192
VECS = CHUNK // 16
NSLOTS = N_EDGES // CHUNK
CHUNKS_PER_TILE = (NSLOTS + NW - 1) // NW
TAIL = N_EDGES - NSLOTS * CHUNK
TAIL_VECS = TAIL // 16

XPAD = N_NODES + 1
XWORDS = XPAD // 2
SLICE = 1056
OUTP = NW * SLICE

_mesh = plsc.VectorSubcoreMesh(
    core_axis_name="c", subcore_axis_name="s", num_cores=NC, num_subcores=NS
)
_params = pltpu.CompilerParams(needs_layout_passes=False)


@functools.partial(
    pl.kernel,
    out_type=jax.ShapeDtypeStruct((NW * OUTP,), jnp.float32),
    mesh=_mesh,
    compiler_params=_params,
    scratch_types=[
        pltpu.VMEM((XWORDS,), jnp.int32),
        pltpu.VMEM((OUTP,), jnp.float32),
        pltpu.VMEM((2, CHUNK), jnp.int32),
        pltpu.VMEM((2, CHUNK), jnp.int32),
        pltpu.VMEM((TAIL,), jnp.int32),
        pltpu.VMEM((TAIL,), jnp.int32),
        pltpu.SemaphoreType.DMA,
        pltpu.SemaphoreType.DMA,
    ],
)
def _edge_accumulate(xpk_hbm, ei_hbm, tsrc_hbm, tdst_hbm, parts_hbm,
                     xpk_v, acc_v, eb0, eb1, tsrc_v, tdst_v, sem0, sem1):
    wid = lax.axis_index("s") * NC + lax.axis_index("c")
    pltpu.sync_copy(xpk_hbm, xpk_v)

    def _zero(i, c):
        acc_v[pl.ds(i * 16, 16)] = jnp.zeros((16,), jnp.float32)
        return c

    lax.fori_loop(0, OUTP // 16, _zero, 0, unroll=8)

    third = jnp.float32(1.0 / 3.0)

    def _body(s, d):
        w = plsc.load_gather(xpk_v, [jnp.right_shift(s, 1)])
        sh = jnp.left_shift(jnp.bitwise_and(s, 1), 4)
        bits = jnp.left_shift(lax.shift_right_logical(w, sh), 16)
        val = plsc.bitcast(bits, jnp.float32)
        q = (d.astype(jnp.float32) * third).astype(jnp.int32)
        keep = (q * 3) == d
        plsc.addupdate_scatter(acc_v, [q], val, mask=keep)

    def _start(slot, buf, sem):
        base = pl.multiple_of(slot * CHUNK, CHUNK)
        pltpu.async_copy(ei_hbm.at[:, pl.ds(base, CHUNK)], buf, sem)

    bufs = (eb0, eb1)
    sems = (sem0, sem1)
    _start(wid, eb0, sem0)
    for ci in range(CHUNKS_PER_TILE):
        cur, csem = bufs[ci % 2], sems[ci % 2]
        if ci + 1 < CHUNKS_PER_TILE:
            nslot = (ci + 1) * NW + wid

            @pl.when(nslot < NSLOTS)
            def _(nslot=nslot, ci=ci):
                _start(nslot, bufs[(ci + 1) % 2], sems[(ci + 1) % 2])

        @pl.when(ci * NW + wid < NSLOTS)
        def _(cur=cur, csem=csem):
            pltpu.make_async_copy(
                ei_hbm.at[:, pl.ds(0, CHUNK)], cur, csem
            ).wait()

            @plsc.parallel_loop(0, CHUNK, step=16, unroll=4)
            def _vec(o):
                _body(cur[0, pl.ds(o, 16)], cur[1, pl.ds(o, 16)])

    @pl.when(wid == NW - 1)
    def _():
        pltpu.sync_copy(tsrc_hbm, tsrc_v)
        pltpu.sync_copy(tdst_hbm, tdst_v)

        @plsc.parallel_loop(0, TAIL, step=16, unroll=4)
        def _tvec(o):
            _body(tsrc_v[pl.ds(o, 16)], tdst_v[pl.ds(o, 16)])

    pltpu.sync_copy(acc_v, parts_hbm.at[pl.ds(wid * OUTP, OUTP)])


@functools.partial(
    pl.kernel,
    out_type=jax.ShapeDtypeStruct((OUTP,), jnp.float32),
    mesh=_mesh,
    compiler_params=_params,
    scratch_types=[
        pltpu.VMEM((NW * SLICE,), jnp.float32),
        pltpu.VMEM((SLICE,), jnp.float32),
        pltpu.VMEM((16,), jnp.float32),
        pltpu.VMEM((16,), jnp.float32),
        pltpu.VMEM((SLICE,), jnp.float32),
        pltpu.SemaphoreType.DMA,
    ],
)
def _combine(parts_hbm, x3_hbm, wl_hbm, wr_hbm, out_hbm,
             rows_v, x3_v, wl_v, wr_v, o_v, sem):
    wid = lax.axis_index("s") * NC + lax.axis_index("c")
    base = wid * SLICE
    cps = [
        pltpu.async_copy(
            parts_hbm.at[pl.ds(r * OUTP + base, SLICE)],
            rows_v.at[pl.ds(r * SLICE, SLICE)],
            sem,
        )
        for r in range(NW)
    ]
    pltpu.sync_copy(x3_hbm.at[pl.ds(base, SLICE)], x3_v)
    pltpu.sync_copy(wl_hbm, wl_v)
    pltpu.sync_copy(wr_hbm, wr_v)
    for cp in cps:
        cp.wait()
    wl = wl_v[...]
    wr = wr_v[...]

    def _red(j, c):
        o = j * 16
        tot = rows_v[pl.ds(o, 16)]
        for r in range(1, NW):
            tot = tot + rows_v[pl.ds(r * SLICE + o, 16)]
        o_v[pl.ds(o, 16)] = tot * wl + x3_v[pl.ds(o, 16)] * wr
        return c

    lax.fori_loop(0, SLICE // 16, _red, 0)
    pltpu.sync_copy(o_v, out_hbm.at[pl.ds(base, SLICE)])


def kernel(x, edge_index, W_l, W_r):
    xf = x.reshape(-1)
    xb = jnp.concatenate(
        [xf.astype(jnp.bfloat16), jnp.zeros((XPAD - N_NODES,), jnp.bfloat16)]
    )
    xpk = lax.bitcast_convert_type(xb.reshape(XWORDS, 2), jnp.int32)
    x3 = jnp.concatenate([xf[::3], jnp.zeros((OUTP - N_OUT,), jnp.float32)])
    wl = jnp.full((16,), W_l[0, 0], jnp.float32)
    wr = jnp.full((16,), W_r[0, 0], jnp.float32)
    ei = edge_index.astype(jnp.int32)
    tsrc = ei[0, N_EDGES - TAIL:]
    tdst = ei[1, N_EDGES - TAIL:]
    parts = _edge_accumulate(xpk, ei, tsrc, tdst)
    outp = _combine(parts, x3, wl, wr)
    return outp[:N_OUT]


# speedup vs baseline: 701.7805x; 1.2672x over previous
import functools

import jax
import jax.numpy as jnp
from jax import lax
from jax.experimental import pallas as pl
from jax.experimental.pallas import tpu as pltpu
from jax.experimental.pallas import tpu_sc as plsc

N_NODES = 99999
N_EDGES = N_NODES * 64
N_OUT = N_NODES // 3

NC, NS = 2, 16
NW = NC * NS

CHUNK = 8192
VECS = CHUNK // 16
NSLOTS = N_EDGES // CHUNK
CHUNKS_PER_TILE = (NSLOTS + NW - 1) // NW
TAIL = N_EDGES - NSLOTS * CHUNK
TAIL_VECS = TAIL // 16

XPAD = N_NODES + 1
XWORDS = XPAD // 2
SLICE = 1056
OUTP = NW * SLICE

_mesh = plsc.VectorSubcoreMesh(
    core_axis_name="c", subcore_axis_name="s", num_cores=NC, num_subcores=NS
)
_params = pltpu.CompilerParams(needs_layout_passes=False)


@functools.partial(
    pl.kernel,
    out_type=jax.ShapeDtypeStruct((NW * OUTP,), jnp.float32),
    mesh=_mesh,
    compiler_params=_params,
    scratch_types=[
        pltpu.VMEM((XWORDS,), jnp.int32),
        pltpu.VMEM((OUTP,), jnp.float32),
        pltpu.VMEM((2, CHUNK), jnp.int32),
        pltpu.VMEM((2, CHUNK), jnp.int32),
        pltpu.VMEM((TAIL,), jnp.int32),
        pltpu.VMEM((TAIL,), jnp.int32),
        pltpu.SemaphoreType.DMA,
        pltpu.SemaphoreType.DMA,
    ],
)
def _edge_accumulate(xpk_hbm, ei_hbm, tsrc_hbm, tdst_hbm, parts_hbm,
                     xpk_v, acc_v, eb0, eb1, tsrc_v, tdst_v, sem0, sem1):
    wid = lax.axis_index("s") * NC + lax.axis_index("c")
    pltpu.sync_copy(xpk_hbm, xpk_v)

    def _zero(i, c):
        acc_v[pl.ds(i * 16, 16)] = jnp.zeros((16,), jnp.float32)
        return c

    lax.fori_loop(0, OUTP // 16, _zero, 0, unroll=8)

    third = jnp.float32(1.0 / 3.0)

    def _body(s, d):
        w = plsc.load_gather(xpk_v, [jnp.right_shift(s, 1)])
        sh = jnp.left_shift(jnp.bitwise_and(s, 1), 4)
        bits = jnp.left_shift(lax.shift_right_logical(w, sh), 16)
        val = plsc.bitcast(bits, jnp.float32)
        q = (d.astype(jnp.float32) * third).astype(jnp.int32)
        keep = (q * 3) == d
        plsc.addupdate_scatter(acc_v, [q], val, mask=keep)

    def _start(slot, buf, sem):
        base = pl.multiple_of(slot * CHUNK, CHUNK)
        pltpu.async_copy(ei_hbm.at[:, pl.ds(base, CHUNK)], buf, sem)

    bufs = (eb0, eb1)
    sems = (sem0, sem1)
    _start(wid, eb0, sem0)
    for ci in range(CHUNKS_PER_TILE):
        cur, csem = bufs[ci % 2], sems[ci % 2]
        if ci + 1 < CHUNKS_PER_TILE:
            nslot = (ci + 1) * NW + wid

            @pl.when(nslot < NSLOTS)
            def _(nslot=nslot, ci=ci):
                _start(nslot, bufs[(ci + 1) % 2], sems[(ci + 1) % 2])

        @pl.when(ci * NW + wid < NSLOTS)
        def _(cur=cur, csem=csem):
            pltpu.make_async_copy(
                ei_hbm.at[:, pl.ds(0, CHUNK)], cur, csem
            ).wait()

            @plsc.parallel_loop(0, CHUNK, step=16, unroll=4)
            def _vec(o):
                _body(cur[0, pl.ds(o, 16)], cur[1, pl.ds(o, 16)])

    @pl.when(wid == NW - 1)
    def _():
        pltpu.sync_copy(tsrc_hbm, tsrc_v)
        pltpu.sync_copy(tdst_hbm, tdst_v)

        @plsc.parallel_loop(0, TAIL, step=16, unroll=4)
        def _tvec(o):
            _body(tsrc_v[pl.ds(o, 16)], tdst_v[pl.ds(o, 16)])

    pltpu.sync_copy(acc_v, parts_hbm.at[pl.ds(wid * OUTP, OUTP)])


@functools.partial(
    pl.kernel,
    out_type=jax.ShapeDtypeStruct((OUTP,), jnp.float32),
    mesh=_mesh,
    compiler_params=_params,
    scratch_types=[
        pltpu.VMEM((NW * SLICE,), jnp.float32),
        pltpu.VMEM((SLICE,), jnp.float32),
        pltpu.VMEM((16,), jnp.float32),
        pltpu.VMEM((16,), jnp.float32),
        pltpu.VMEM((SLICE,), jnp.float32),
        pltpu.SemaphoreType.DMA,
    ],
)
def _combine(parts_hbm, x3_hbm, wl_hbm, wr_hbm, out_hbm,
             rows_v, x3_v, wl_v, wr_v, o_v, sem):
    wid = lax.axis_index("s") * NC + lax.axis_index("c")
    base = wid * SLICE
    cps = [
        pltpu.async_copy(
            parts_hbm.at[pl.ds(r * OUTP + base, SLICE)],
            rows_v.at[pl.ds(r * SLICE, SLICE)],
            sem,
        )
        for r in range(NW)
    ]
    pltpu.sync_copy(x3_hbm.at[pl.ds(base, SLICE)], x3_v)
    pltpu.sync_copy(wl_hbm, wl_v)
    pltpu.sync_copy(wr_hbm, wr_v)
    for cp in cps:
        cp.wait()
    wl = wl_v[...]
    wr = wr_v[...]

    def _red(j, c):
        o = j * 16
        tot = rows_v[pl.ds(o, 16)]
        for r in range(1, NW):
            tot = tot + rows_v[pl.ds(r * SLICE + o, 16)]
        o_v[pl.ds(o, 16)] = tot * wl + x3_v[pl.ds(o, 16)] * wr
        return c

    lax.fori_loop(0, SLICE // 16, _red, 0)
    pltpu.sync_copy(o_v, out_hbm.at[pl.ds(base, SLICE)])


def kernel(x, edge_index, W_l, W_r):
    xf = x.reshape(-1)
    xb = jnp.concatenate(
        [xf.astype(jnp.bfloat16), jnp.zeros((XPAD - N_NODES,), jnp.bfloat16)]
    )
    xu = lax.bitcast_convert_type(xb, jnp.uint16)
    lo = xu[0::2].astype(jnp.uint32)
    hi = xu[1::2].astype(jnp.uint32)
    xpk = lax.bitcast_convert_type(lo | (hi << 16), jnp.int32)
    x3 = jnp.concatenate([xf[::3], jnp.zeros((OUTP - N_OUT,), jnp.float32)])
    wl = jnp.full((16,), W_l[0, 0], jnp.float32)
    wr = jnp.full((16,), W_r[0, 0], jnp.float32)
    ei = edge_index.astype(jnp.int32)
    tsrc = ei[0, N_EDGES - TAIL:]
    tdst = ei[1, N_EDGES - TAIL:]
    parts = _edge_accumulate(xpk, ei, tsrc, tdst)
    outp = _combine(parts, x3, wl, wr)
    return outp[:N_OUT]


# speedup vs baseline: 766.7947x; 1.0926x over previous
import functools

import jax
import jax.numpy as jnp
from jax import lax
from jax.experimental import pallas as pl
from jax.experimental.pallas import tpu as pltpu
from jax.experimental.pallas import tpu_sc as plsc

N_NODES = 99999
N_EDGES = N_NODES * 64
N_OUT = N_NODES // 3

NC, NS = 2, 16
NW = NC * NS

CHUNK = 8192
VECS = CHUNK // 16
NSLOTS = N_EDGES // CHUNK
CHUNKS_PER_TILE = (NSLOTS + NW - 1) // NW
TAIL = N_EDGES - NSLOTS * CHUNK
TAIL_VECS = TAIL // 16

XPAD = N_NODES + 1
XWORDS = XPAD // 2
RBCH = 4096
RB_PIECES = XPAD // RBCH
RB_TAIL = XPAD - RB_PIECES * RBCH
SLICE = 1056
OUTP = NW * SLICE

_mesh = plsc.VectorSubcoreMesh(
    core_axis_name="c", subcore_axis_name="s", num_cores=NC, num_subcores=NS
)
_params = pltpu.CompilerParams(needs_layout_passes=False)


@functools.partial(
    pl.kernel,
    out_type=jax.ShapeDtypeStruct((NW * OUTP,), jnp.float32),
    mesh=_mesh,
    compiler_params=_params,
    scratch_types=[
        pltpu.VMEM((XWORDS,), jnp.int32),
        pltpu.VMEM((OUTP,), jnp.float32),
        pltpu.VMEM((2, CHUNK), jnp.int32),
        pltpu.VMEM((2, CHUNK), jnp.int32),
        pltpu.VMEM((RBCH,), jnp.int32),
        pltpu.VMEM((RBCH,), jnp.int32),
        pltpu.VMEM((TAIL,), jnp.int32),
        pltpu.VMEM((TAIL,), jnp.int32),
        pltpu.SemaphoreType.DMA,
        pltpu.SemaphoreType.DMA,
        pltpu.SemaphoreType.DMA,
        pltpu.SemaphoreType.DMA,
    ],
)
def _edge_accumulate(rb_hbm, ei_hbm, tsrc_hbm, tdst_hbm, parts_hbm,
                     xpk_v, acc_v, eb0, eb1, rb0, rb1, tsrc_v, tdst_v,
                     sem0, sem1, sem2, sem3):
    wid = lax.axis_index("s") * NC + lax.axis_index("c")

    def _start(slot, buf, sem):
        base = pl.multiple_of(slot * CHUNK, CHUNK)
        pltpu.async_copy(ei_hbm.at[:, pl.ds(base, CHUNK)], buf, sem)

    _start(wid, eb0, sem0)

    rbufs, rsems = (rb0, rb1), (sem2, sem3)
    rsizes = [RBCH] * RB_PIECES + [RB_TAIL]
    iota2 = lax.iota(jnp.int32, 16) * 2
    pltpu.async_copy(rb_hbm.at[pl.ds(0, RBCH)], rb0, sem2)
    for p, sz in enumerate(rsizes):
        cur, csem = rbufs[p % 2], rsems[p % 2]
        if p + 1 < len(rsizes):
            nsz = rsizes[p + 1]
            pltpu.async_copy(
                rb_hbm.at[pl.ds((p + 1) * RBCH, nsz)],
                rbufs[(p + 1) % 2].at[pl.ds(0, nsz)],
                rsems[(p + 1) % 2],
            )
        pltpu.make_async_copy(
            rb_hbm.at[pl.ds(0, sz)], cur.at[pl.ds(0, sz)], csem
        ).wait()
        pbase = p * (RBCH // 2)

        @plsc.parallel_loop(0, sz // 32, step=1, unroll=4)
        def _pack(k, cur=cur, pbase=pbase):
            o2 = k * 32
            a = plsc.load_gather(cur, [o2 + iota2])
            b = plsc.load_gather(cur, [o2 + iota2 + 1])
            xpk_v[pl.ds(pbase + k * 16, 16)] = a | jnp.left_shift(b, 16)

    def _zero(i, c):
        acc_v[pl.ds(i * 16, 16)] = jnp.zeros((16,), jnp.float32)
        return c

    lax.fori_loop(0, OUTP // 16, _zero, 0, unroll=8)

    third = jnp.float32(1.0 / 3.0)

    def _body(s, d):
        w = plsc.load_gather(xpk_v, [jnp.right_shift(s, 1)])
        sh = jnp.left_shift(jnp.bitwise_and(s, 1), 4)
        bits = jnp.left_shift(lax.shift_right_logical(w, sh), 16)
        val = plsc.bitcast(bits, jnp.float32)
        q = (d.astype(jnp.float32) * third).astype(jnp.int32)
        keep = (q * 3) == d
        plsc.addupdate_scatter(acc_v, [q], val, mask=keep)

    bufs = (eb0, eb1)
    sems = (sem0, sem1)
    for ci in range(CHUNKS_PER_TILE):
        cur, csem = bufs[ci % 2], sems[ci % 2]
        if ci + 1 < CHUNKS_PER_TILE:
            nslot = (ci + 1) * NW + wid

            @pl.when(nslot < NSLOTS)
            def _(nslot=nslot, ci=ci):
                _start(nslot, bufs[(ci + 1) % 2], sems[(ci + 1) % 2])

        @pl.when(ci * NW + wid < NSLOTS)
        def _(cur=cur, csem=csem):
            pltpu.make_async_copy(
                ei_hbm.at[:, pl.ds(0, CHUNK)], cur, csem
            ).wait()

            @plsc.parallel_loop(0, CHUNK, step=16, unroll=4)
            def _vec(o):
                _body(cur[0, pl.ds(o, 16)], cur[1, pl.ds(o, 16)])

    @pl.when(wid == NW - 1)
    def _():
        pltpu.sync_copy(tsrc_hbm, tsrc_v)
        pltpu.sync_copy(tdst_hbm, tdst_v)

        @plsc.parallel_loop(0, TAIL, step=16, unroll=4)
        def _tvec(o):
            _body(tsrc_v[pl.ds(o, 16)], tdst_v[pl.ds(o, 16)])

    pltpu.sync_copy(acc_v, parts_hbm.at[pl.ds(wid * OUTP, OUTP)])


@functools.partial(
    pl.kernel,
    out_type=jax.ShapeDtypeStruct((OUTP,), jnp.float32),
    mesh=_mesh,
    compiler_params=_params,
    scratch_types=[
        pltpu.VMEM((NW * SLICE,), jnp.float32),
        pltpu.VMEM((SLICE,), jnp.float32),
        pltpu.VMEM((16,), jnp.float32),
        pltpu.VMEM((16,), jnp.float32),
        pltpu.VMEM((SLICE,), jnp.float32),
        pltpu.SemaphoreType.DMA,
    ],
)
def _combine(parts_hbm, x3_hbm, wl_hbm, wr_hbm, out_hbm,
             rows_v, x3_v, wl_v, wr_v, o_v, sem):
    wid = lax.axis_index("s") * NC + lax.axis_index("c")
    base = wid * SLICE
    cps = [
        pltpu.async_copy(
            parts_hbm.at[pl.ds(r * OUTP + base, SLICE)],
            rows_v.at[pl.ds(r * SLICE, SLICE)],
            sem,
        )
        for r in range(NW)
    ]
    pltpu.sync_copy(x3_hbm.at[pl.ds(base, SLICE)], x3_v)
    pltpu.sync_copy(wl_hbm, wl_v)
    pltpu.sync_copy(wr_hbm, wr_v)
    for cp in cps:
        cp.wait()
    wl = wl_v[...]
    wr = wr_v[...]

    def _red(j, c):
        o = j * 16
        tot = rows_v[pl.ds(o, 16)]
        for r in range(1, NW):
            tot = tot + rows_v[pl.ds(r * SLICE + o, 16)]
        o_v[pl.ds(o, 16)] = tot * wl + x3_v[pl.ds(o, 16)] * wr
        return c

    lax.fori_loop(0, SLICE // 16, _red, 0)
    pltpu.sync_copy(o_v, out_hbm.at[pl.ds(base, SLICE)])


def kernel(x, edge_index, W_l, W_r):
    xf = x.reshape(-1)
    xi = lax.bitcast_convert_type(
        jnp.concatenate([xf, jnp.zeros((XPAD - N_NODES,), jnp.float32)]),
        jnp.uint32,
    )
    rbits = (xi + 0x7FFF + ((xi >> 16) & 1)) >> 16
    rb = lax.bitcast_convert_type(rbits, jnp.int32)
    x3 = jnp.concatenate([xf[::3], jnp.zeros((OUTP - N_OUT,), jnp.float32)])
    wl = jnp.full((16,), W_l[0, 0], jnp.float32)
    wr = jnp.full((16,), W_r[0, 0], jnp.float32)
    ei = edge_index.astype(jnp.int32)
    tsrc = ei[0, N_EDGES - TAIL:]
    tdst = ei[1, N_EDGES - TAIL:]
    parts = _edge_accumulate(rb, ei, tsrc, tdst)
    outp = _combine(parts, x3, wl, wr)
    return outp[:N_OUT]
